# baseline (speedup 1.0000x reference)
import functools

import jax
import jax.numpy as jnp
from jax import lax
from jax.experimental import pallas as pl
from jax.experimental.pallas import tpu as pltpu
from jax.experimental.pallas import tpu_sc as plsc

NC = 2
NS = 16
LANES = 16
CHUNK = 128
NP = 2


def _sc_mesh():
  return plsc.VectorSubcoreMesh(core_axis_name="c", subcore_axis_name="s")


def _make_deg_kernel(n, d_half, pad_e, half_rows, acc_rows):
  chunks_per_tile = pad_e // (NC * NS * CHUNK)
  nct = pad_e // CHUNK
  zrows = acc_rows // NS
  orows = 1000
  otiles = half_rows // orows

  @functools.partial(
      pl.kernel,
      out_type=jax.ShapeDtypeStruct((NC * n, d_half), jnp.float32),
      mesh=_sc_mesh(),
      scratch_types=[
          pltpu.VMEM((CHUNK, d_half), jnp.float32),
          pltpu.VMEM((chunks_per_tile, CHUNK), jnp.int32),
          pltpu.VMEM_SHARED((acc_rows, d_half), jnp.float32),
      ],
  )
  def deg_kernel(dstp_hbm, ones_hbm, zeros_hbm, out_hbm, ones_v, didx_v, acc_sh):
    c = lax.axis_index("c")
    s = lax.axis_index("s")
    pltpu.sync_copy(ones_hbm, ones_v)
    for p in range(NP):
      pltpu.sync_copy(zeros_hbm.at[pl.ds(s * zrows, zrows)],
                      acc_sh.at[pl.ds(s * zrows, zrows)])
      pltpu.sync_copy(
          dstp_hbm.at[pl.ds(p * nct + c * (nct // NC) + s * chunks_per_tile,
                            chunks_per_tile)],
          didx_v)
      plsc.subcore_barrier()

      def body(i, carry):
        pltpu.sync_copy(ones_v, acc_sh.at[didx_v.at[i]], add=True)
        return carry

      lax.fori_loop(0, chunks_per_tile, body, 0)
      plsc.subcore_barrier()

      @pl.when(s < otiles)
      def _():
        pltpu.sync_copy(
            acc_sh.at[pl.ds(s * orows, orows)],
            out_hbm.at[pl.ds(c * n + p * half_rows + s * orows, orows)])

      plsc.subcore_barrier()

  return deg_kernel


def _make_agg_kernel(n, d_half, pad_e, half_rows, acc_rows):
  chunks_per_tile = pad_e // (NS * CHUNK)
  nct = pad_e // CHUNK
  zrows = acc_rows // NS
  orows = 1000
  otiles = half_rows // orows

  nbuf = 4
  grp = 4
  ngrp = chunks_per_tile // grp
  assert chunks_per_tile % (4 * grp) == 0

  @functools.partial(
      pl.kernel,
      out_type=jax.ShapeDtypeStruct((NC * n, d_half), jnp.float32),
      mesh=_sc_mesh(),
      scratch_types=[
          pltpu.VMEM((chunks_per_tile, CHUNK), jnp.int32),
          pltpu.VMEM((4 * grp, CHUNK), jnp.int32),
          pltpu.VMEM((nbuf, CHUNK, d_half), jnp.float32),
          pltpu.VMEM_SHARED((acc_rows, d_half), jnp.float32),
      ] + [pltpu.SemaphoreType.DMA] * (2 * nbuf + 4),
  )
  def agg_kernel(table_hbm, src2_hbm, dstp_hbm, zeros_hbm, out_hbm,
                 sidx_v, didx_v, rows_v, acc_sh, *sems):
    gsems = sems[:nbuf]
    ssems = sems[nbuf:2 * nbuf]
    dsems = sems[2 * nbuf:]
    c = lax.axis_index("c")
    s = lax.axis_index("s")
    pltpu.sync_copy(
        src2_hbm.at[pl.ds(c * nct + s * chunks_per_tile, chunks_per_tile)],
        sidx_v)
    for p in range(NP):
      pltpu.sync_copy(zeros_hbm.at[pl.ds(s * zrows, zrows)],
                      acc_sh.at[pl.ds(s * zrows, zrows)])
      dbase = p * nct + s * chunks_per_tile
      pltpu.sync_copy(dstp_hbm.at[pl.ds(dbase, 2 * grp)],
                      didx_v.at[pl.ds(0, 2 * grp)])
      for b in range(2):
        pltpu.async_copy(table_hbm.at[sidx_v.at[b]], rows_v.at[b], gsems[b])
      plsc.subcore_barrier()

      def outer(ooo, carry):
        for qq in range(4):
          o = ooo * 4 + qq
          @pl.when(o + 2 < ngrp)
          def _():
            pltpu.async_copy(
                dstp_hbm.at[pl.ds(dbase + (o + 2) * grp, grp)],
                didx_v.at[pl.ds(((qq + 2) % 4) * grp, grp)],
                dsems[(qq + 2) % 4])

          @pl.when(o > 1)
          def _():
            pltpu.make_async_copy(
                dstp_hbm.at[pl.ds(dbase + o * grp, grp)],
                didx_v.at[pl.ds(qq * grp, grp)], dsems[qq]).wait()

          for b in range(grp):
            i = o * grp + b
            bb = b
            pltpu.make_async_copy(table_hbm.at[sidx_v.at[i]], rows_v.at[bb],
                                  gsems[bb]).wait()
            pltpu.async_copy(rows_v.at[bb], acc_sh.at[didx_v.at[qq * grp + b]],
                             ssems[bb], add=True)

            @pl.when(i > 1)
            def _():
              pltpu.make_async_copy(
                  rows_v.at[(bb + 2) % nbuf],
                  acc_sh.at[didx_v.at[qq * grp + b]],
                  ssems[(bb + 2) % nbuf]).wait()

            @pl.when(i + 2 < chunks_per_tile)
            def _():
              pltpu.async_copy(table_hbm.at[sidx_v.at[i + 2]],
                               rows_v.at[(bb + 2) % nbuf],
                               gsems[(bb + 2) % nbuf])

        return carry

      lax.fori_loop(0, ngrp // 4, outer, 0)
      for j in (chunks_per_tile - 2, chunks_per_tile - 1):
        pltpu.make_async_copy(rows_v.at[j % nbuf], acc_sh.at[didx_v.at[0]],
                              ssems[j % nbuf]).wait()
      plsc.subcore_barrier()

      @pl.when(s < otiles)
      def _():
        pltpu.sync_copy(
            acc_sh.at[pl.ds(s * orows, orows)],
            out_hbm.at[pl.ds(c * n + p * half_rows + s * orows, orows)])

      plsc.subcore_barrier()

  return agg_kernel


def _dinv_from_degp(degp):
  deg = degp[0, :, 0:1] + degp[1, :, 0:1] + 1.0
  return lax.rsqrt(deg)


def _split_h(u, u_ref):
  dh = u.shape[1] // NC
  for q in range(NC):
    u_ref[q] = u[:, q * dh:(q + 1) * dh]


def _cat_h(acc_ref, uin_ref):
  return jnp.concatenate([acc_ref[q] + uin_ref[q] for q in range(NC)], axis=1)


def _tc_first_body(x_ref, w_ref, degp_ref, u_ref):
  dinv = _dinv_from_degp(degp_ref[...])
  g = jnp.dot(x_ref[...], w_ref[...], preferred_element_type=jnp.float32)
  _split_h(g * dinv, u_ref)


def _tc_mid_body(acc_ref, uin_ref, b_ref, w_ref, degp_ref, u_ref):
  dinv = _dinv_from_degp(degp_ref[...])
  h = jnp.maximum(_cat_h(acc_ref, uin_ref) * dinv + b_ref[...], 0.0)
  g = jnp.dot(h, w_ref[...], preferred_element_type=jnp.float32)
  _split_h(g * dinv, u_ref)


def _tc_last_body(acc_ref, uin_ref, b_ref, w_ref, bfc_ref, degp_ref, o_ref):
  dinv = _dinv_from_degp(degp_ref[...])
  h = jnp.maximum(_cat_h(acc_ref, uin_ref) * dinv + b_ref[...], 0.0)
  o_ref[...] = (jnp.dot(h, w_ref[...], preferred_element_type=jnp.float32)
                + bfc_ref[...])


def _row_spec(r, cols):
  return pl.BlockSpec((r, cols), lambda i: (i, 0))


def _stack_spec(lead, r, cols):
  return pl.BlockSpec((lead, r, cols), lambda i: (0, i, 0))


def _full_spec(shape):
  return pl.BlockSpec(shape, lambda i: tuple(0 for _ in shape))


def kernel(x, edge_index, W1, b1, W2, b2, W3, b3, Wfc, bfc):
  n, d_in = x.shape
  hid = W1.shape[1]
  d_half = hid // NC
  n_cls = Wfc.shape[1]
  e = edge_index.shape[1]

  slot = NC * NS * CHUNK * 8
  pad_e = ((e + slot - 1) // slot) * slot
  half_rows = n // NP
  acc_rows = ((half_rows + 64 + 127) // 128) * 128
  nct = pad_e // CHUNK

  src = edge_index[0].astype(jnp.int32)
  dst = edge_index[1].astype(jnp.int32)
  pad = pad_e - e
  src = jnp.concatenate([src, jnp.zeros((pad,), jnp.int32)])
  dst = jnp.concatenate([dst, jnp.full((pad,), n, jnp.int32)])
  src2 = (src[None, :] + (jnp.arange(NC, dtype=jnp.int32) * n)[:, None])
  src2 = src2.reshape(NC * nct, CHUNK)
  trash = half_rows + (jnp.arange(pad_e, dtype=jnp.int32) % 64)
  local = dst[None, :] - (jnp.arange(NP, dtype=jnp.int32) * half_rows)[:, None]
  dstp = jnp.where((local >= 0) & (local < half_rows), local, trash[None, :])
  dstp = dstp.reshape(NP * nct, CHUNK)

  ones_rows = jnp.ones((CHUNK, d_half), jnp.float32)
  zeros_acc = jnp.zeros((acc_rows, d_half), jnp.float32)

  deg_kernel = _make_deg_kernel(n, d_half, pad_e, half_rows, acc_rows)
  agg_kernel = _make_agg_kernel(n, d_half, pad_e, half_rows, acc_rows)

  degp = deg_kernel(dstp, ones_rows, zeros_acc)
  degp = degp.reshape(NC, n, d_half)

  r = 1000
  grid = (n // r,)

  u1 = pl.pallas_call(
      _tc_first_body,
      grid=grid,
      in_specs=[_row_spec(r, d_in), _full_spec((d_in, hid)),
                _stack_spec(NC, r, d_half)],
      out_specs=_stack_spec(NC, r, d_half),
      out_shape=jax.ShapeDtypeStruct((NC, n, d_half), jnp.float32),
  )(x, W1, degp)

  def mid(u_prev, b_prev, w_next):
    acc = agg_kernel(u_prev.reshape(NC * n, d_half), src2, dstp, zeros_acc)
    return pl.pallas_call(
        _tc_mid_body,
        grid=grid,
        in_specs=[_stack_spec(NC, r, d_half), _stack_spec(NC, r, d_half),
                  _full_spec((1, hid)), _full_spec((hid, hid)),
                  _stack_spec(NC, r, d_half)],
        out_specs=_stack_spec(NC, r, d_half),
        out_shape=jax.ShapeDtypeStruct((NC, n, d_half), jnp.float32),
    )(acc.reshape(NC, n, d_half), u_prev, b_prev.reshape(1, hid), w_next,
      degp)

  u2 = mid(u1, b1, W2)
  u3 = mid(u2, b2, W3)

  acc3 = agg_kernel(u3.reshape(NC * n, d_half), src2, dstp, zeros_acc)
  out = pl.pallas_call(
      _tc_last_body,
      grid=grid,
      in_specs=[_stack_spec(NC, r, d_half), _stack_spec(NC, r, d_half),
                _full_spec((1, hid)), _full_spec((hid, n_cls)),
                _full_spec((1, n_cls)), _stack_spec(NC, r, d_half)],
      out_specs=_row_spec(r, n_cls),
      out_shape=jax.ShapeDtypeStruct((n, n_cls), jnp.float32),
  )(acc3.reshape(NC, n, d_half), u3, b3.reshape(1, hid), Wfc,
    bfc.reshape(1, n_cls), degp)
  return out


# speedup vs baseline: 2.0474x; 2.0474x over previous
import functools

import jax
import jax.numpy as jnp
from jax import lax
from jax.experimental import pallas as pl
from jax.experimental.pallas import tpu as pltpu
from jax.experimental.pallas import tpu_sc as plsc

NC = 2
NS = 16
CHUNK = 128
GRP = 4


def _sc_mesh():
  return plsc.VectorSubcoreMesh(core_axis_name="c", subcore_axis_name="s")


def _make_deg_kernel(n, d_half, pad_e, acc_rows):
  chunks_per_tile = pad_e // (NC * NS * CHUNK)
  zrows = acc_rows // NS
  orows = 1000
  otiles = n // orows

  @functools.partial(
      pl.kernel,
      out_type=jax.ShapeDtypeStruct((NC * n, d_half), jnp.float32),
      mesh=_sc_mesh(),
      scratch_types=[
          pltpu.VMEM((CHUNK, d_half), jnp.float32),
          pltpu.VMEM((chunks_per_tile, CHUNK), jnp.int32),
          pltpu.VMEM_SHARED((acc_rows, d_half), jnp.float32),
      ],
  )
  def deg_kernel(dst_hbm, ones_hbm, zeros_hbm, out_hbm, ones_v, didx_v, acc_sh):
    c = lax.axis_index("c")
    s = lax.axis_index("s")
    w = c * NS + s
    pltpu.sync_copy(ones_hbm, ones_v)
    pltpu.sync_copy(zeros_hbm.at[pl.ds(s * zrows, zrows)],
                    acc_sh.at[pl.ds(s * zrows, zrows)])
    pltpu.sync_copy(dst_hbm.at[pl.ds(w * chunks_per_tile, chunks_per_tile)],
                    didx_v)
    plsc.subcore_barrier()

    def body(i, carry):
      pltpu.sync_copy(ones_v, acc_sh.at[didx_v.at[i]], add=True)
      return carry

    lax.fori_loop(0, chunks_per_tile, body, 0)
    plsc.subcore_barrier()

    @pl.when(s < otiles)
    def _():
      pltpu.sync_copy(acc_sh.at[pl.ds(s * orows, orows)],
                      out_hbm.at[pl.ds(c * n + s * orows, orows)])

  return deg_kernel


def _make_agg_kernel(n, d_half, pad_e, acc_rows):
  chunks_per_tile = pad_e // (NS * CHUNK)
  nct = pad_e // CHUNK
  ngrp = chunks_per_tile // GRP
  assert chunks_per_tile % (4 * GRP) == 0
  zrows = acc_rows // NS
  orows = 1000
  otiles = n // orows

  @functools.partial(
      pl.kernel,
      out_type=jax.ShapeDtypeStruct((NC * n, d_half), jnp.float32),
      mesh=_sc_mesh(),
      scratch_types=[
          pltpu.VMEM((4 * GRP, CHUNK), jnp.int32),
          pltpu.VMEM((4 * GRP, CHUNK), jnp.int32),
          pltpu.VMEM((2, CHUNK, d_half), jnp.float32),
          pltpu.VMEM_SHARED((acc_rows, d_half), jnp.float32),
      ] + [pltpu.SemaphoreType.DMA] * 12,
  )
  def agg_kernel(table_hbm, src2_hbm, dst_hbm, zeros_hbm, out_hbm,
                 sidx_v, didx_v, rows_v, acc_sh, *sems):
    gsems = sems[0:2]
    ssems = sems[2:4]
    sisems = sems[4:8]
    disems = sems[8:12]
    c = lax.axis_index("c")
    s = lax.axis_index("s")
    sbase = c * nct + s * chunks_per_tile
    dbase = s * chunks_per_tile
    pltpu.sync_copy(zeros_hbm.at[pl.ds(s * zrows, zrows)],
                    acc_sh.at[pl.ds(s * zrows, zrows)])
    pltpu.sync_copy(src2_hbm.at[pl.ds(sbase, 2 * GRP)],
                    sidx_v.at[pl.ds(0, 2 * GRP)])
    pltpu.sync_copy(dst_hbm.at[pl.ds(dbase, 2 * GRP)],
                    didx_v.at[pl.ds(0, 2 * GRP)])
    pltpu.async_copy(table_hbm.at[sidx_v.at[0]], rows_v.at[0], gsems[0])
    plsc.subcore_barrier()

    def outer(ooo, carry):
      for qq in range(4):
        o = ooo * 4 + qq
        @pl.when(o + 2 < ngrp)
        def _():
          sl = (qq + 2) % 4
          pltpu.async_copy(src2_hbm.at[pl.ds(sbase + (o + 2) * GRP, GRP)],
                           sidx_v.at[pl.ds(sl * GRP, GRP)], sisems[sl])
          pltpu.async_copy(dst_hbm.at[pl.ds(dbase + (o + 2) * GRP, GRP)],
                           didx_v.at[pl.ds(sl * GRP, GRP)], disems[sl])

        @pl.when(jnp.logical_and(o >= 1, o + 1 < ngrp))
        def _():
          sl = (qq + 1) % 4
          pltpu.make_async_copy(src2_hbm.at[pl.ds(sbase + (o + 1) * GRP, GRP)],
                                sidx_v.at[pl.ds(sl * GRP, GRP)],
                                sisems[sl]).wait()
          pltpu.make_async_copy(dst_hbm.at[pl.ds(dbase + (o + 1) * GRP, GRP)],
                                didx_v.at[pl.ds(sl * GRP, GRP)],
                                disems[sl]).wait()

        for b4 in range(GRP):
          i = o * GRP + b4
          b = b4 & 1

          @pl.when(i > 0)
          def _():
            pltpu.make_async_copy(rows_v.at[1 - b],
                                  acc_sh.at[didx_v.at[qq * GRP + b4]],
                                  ssems[1 - b]).wait()

          @pl.when(i + 1 < chunks_per_tile)
          def _():
            pltpu.async_copy(table_hbm.at[sidx_v.at[(qq * GRP + b4 + 1) % 16]],
                             rows_v.at[1 - b], gsems[1 - b])

          pltpu.make_async_copy(table_hbm.at[sidx_v.at[qq * GRP + b4]],
                                rows_v.at[b], gsems[b]).wait()
          pltpu.async_copy(rows_v.at[b], acc_sh.at[didx_v.at[qq * GRP + b4]],
                           ssems[b], add=True)

      return carry

    lax.fori_loop(0, ngrp // 4, outer, 0)
    pltpu.make_async_copy(rows_v.at[(chunks_per_tile - 1) % 2],
                          acc_sh.at[didx_v.at[0]],
                          ssems[(chunks_per_tile - 1) % 2]).wait()
    plsc.subcore_barrier()

    @pl.when(s < otiles)
    def _():
      pltpu.sync_copy(acc_sh.at[pl.ds(s * orows, orows)],
                      out_hbm.at[pl.ds(c * n + s * orows, orows)])

  return agg_kernel


def _dinv_from_degp(degp):
  deg = degp[0, :, 0:1] + degp[1, :, 0:1] + 1.0
  return lax.rsqrt(deg)


def _split_h(u, u_ref):
  dh = u.shape[1] // NC
  for q in range(NC):
    u_ref[q] = u[:, q * dh:(q + 1) * dh]


def _cat_h(acc_ref, uin_ref):
  return jnp.concatenate([acc_ref[q] + uin_ref[q] for q in range(NC)], axis=1)


def _tc_first_body(x_ref, w_ref, degp_ref, u_ref):
  dinv = _dinv_from_degp(degp_ref[...])
  g = jnp.dot(x_ref[...], w_ref[...], preferred_element_type=jnp.float32)
  _split_h(g * dinv, u_ref)


def _tc_mid_body(acc_ref, uin_ref, b_ref, w_ref, degp_ref, u_ref):
  dinv = _dinv_from_degp(degp_ref[...])
  h = jnp.maximum(_cat_h(acc_ref, uin_ref) * dinv + b_ref[...], 0.0)
  g = jnp.dot(h, w_ref[...], preferred_element_type=jnp.float32)
  _split_h(g * dinv, u_ref)


def _tc_last_body(acc_ref, uin_ref, b_ref, w_ref, bfc_ref, degp_ref, o_ref):
  dinv = _dinv_from_degp(degp_ref[...])
  h = jnp.maximum(_cat_h(acc_ref, uin_ref) * dinv + b_ref[...], 0.0)
  o_ref[...] = (jnp.dot(h, w_ref[...], preferred_element_type=jnp.float32)
                + bfc_ref[...])


def _row_spec(r, cols):
  return pl.BlockSpec((r, cols), lambda i: (i, 0))


def _stack_spec(lead, r, cols):
  return pl.BlockSpec((lead, r, cols), lambda i: (0, i, 0))


def _full_spec(shape):
  return pl.BlockSpec(shape, lambda i: tuple(0 for _ in shape))


def kernel(x, edge_index, W1, b1, W2, b2, W3, b3, Wfc, bfc):
  n, d_in = x.shape
  hid = W1.shape[1]
  d_half = hid // NC
  n_cls = Wfc.shape[1]
  e = edge_index.shape[1]

  slot = NC * NS * CHUNK * 4 * GRP
  pad_e = ((e + slot - 1) // slot) * slot
  acc_rows = ((n + 1 + 127) // 128) * 128
  nct = pad_e // CHUNK

  src = edge_index[0].astype(jnp.int32)
  dst = edge_index[1].astype(jnp.int32)
  pad = pad_e - e
  src = jnp.concatenate([src, jnp.zeros((pad,), jnp.int32)])
  dst = jnp.concatenate([dst, jnp.full((pad,), n, jnp.int32)])
  src2 = (src[None, :] + (jnp.arange(NC, dtype=jnp.int32) * n)[:, None])
  src2 = src2.reshape(NC * nct, CHUNK)
  dst2 = dst.reshape(nct, CHUNK)

  ones_rows = jnp.ones((CHUNK, d_half), jnp.float32)
  zeros_acc = jnp.zeros((acc_rows, d_half), jnp.float32)

  deg_kernel = _make_deg_kernel(n, d_half, pad_e, acc_rows)
  agg_kernel = _make_agg_kernel(n, d_half, pad_e, acc_rows)

  degp = deg_kernel(dst2, ones_rows, zeros_acc)
  degp = degp.reshape(NC, n, d_half)

  r = 1000
  grid = (n // r,)

  u1 = pl.pallas_call(
      _tc_first_body,
      grid=grid,
      in_specs=[_row_spec(r, d_in), _full_spec((d_in, hid)),
                _stack_spec(NC, r, d_half)],
      out_specs=_stack_spec(NC, r, d_half),
      out_shape=jax.ShapeDtypeStruct((NC, n, d_half), jnp.float32),
  )(x, W1, degp)

  def mid(u_prev, b_prev, w_next):
    acc = agg_kernel(u_prev.reshape(NC * n, d_half), src2, dst2, zeros_acc)
    return pl.pallas_call(
        _tc_mid_body,
        grid=grid,
        in_specs=[_stack_spec(NC, r, d_half), _stack_spec(NC, r, d_half),
                  _full_spec((1, hid)), _full_spec((hid, hid)),
                  _stack_spec(NC, r, d_half)],
        out_specs=_stack_spec(NC, r, d_half),
        out_shape=jax.ShapeDtypeStruct((NC, n, d_half), jnp.float32),
    )(acc.reshape(NC, n, d_half), u_prev, b_prev.reshape(1, hid), w_next,
      degp)

  u2 = mid(u1, b1, W2)
  u3 = mid(u2, b2, W3)

  acc3 = agg_kernel(u3.reshape(NC * n, d_half), src2, dst2, zeros_acc)
  out = pl.pallas_call(
      _tc_last_body,
      grid=grid,
      in_specs=[_stack_spec(NC, r, d_half), _stack_spec(NC, r, d_half),
                _full_spec((1, hid)), _full_spec((hid, n_cls)),
                _full_spec((1, n_cls)), _stack_spec(NC, r, d_half)],
      out_specs=_row_spec(r, n_cls),
      out_shape=jax.ShapeDtypeStruct((n, n_cls), jnp.float32),
  )(acc3.reshape(NC, n, d_half), u3, b3.reshape(1, hid), Wfc,
    bfc.reshape(1, n_cls), degp)
  return out


# speedup vs baseline: 2.0690x; 1.0106x over previous
import functools

import jax
import jax.numpy as jnp
from jax import lax
from jax.experimental import pallas as pl
from jax.experimental.pallas import tpu as pltpu
from jax.experimental.pallas import tpu_sc as plsc

NC = 2
NS = 16
CHUNK = 128
GRP = 4


def _sc_mesh():
  return plsc.VectorSubcoreMesh(core_axis_name="c", subcore_axis_name="s")


def _make_deg_kernel(n, d_half, pad_e, acc_rows):
  chunks_per_tile = pad_e // (NC * NS * CHUNK)
  zrows = acc_rows // NS
  orows = 1000
  otiles = n // orows

  @functools.partial(
      pl.kernel,
      out_type=jax.ShapeDtypeStruct((NC * n, d_half), jnp.float32),
      mesh=_sc_mesh(),
      scratch_types=[
          pltpu.VMEM((CHUNK, d_half), jnp.float32),
          pltpu.VMEM((chunks_per_tile, CHUNK), jnp.int32),
          pltpu.VMEM_SHARED((acc_rows, d_half), jnp.float32),
      ],
  )
  def deg_kernel(dst_hbm, ones_hbm, zeros_hbm, out_hbm, ones_v, didx_v, acc_sh):
    c = lax.axis_index("c")
    s = lax.axis_index("s")
    w = c * NS + s
    pltpu.sync_copy(ones_hbm, ones_v)
    pltpu.sync_copy(zeros_hbm.at[pl.ds(s * zrows, zrows)],
                    acc_sh.at[pl.ds(s * zrows, zrows)])
    pltpu.sync_copy(dst_hbm.at[pl.ds(w * chunks_per_tile, chunks_per_tile)],
                    didx_v)
    plsc.subcore_barrier()

    def body(i, carry):
      pltpu.sync_copy(ones_v, acc_sh.at[didx_v.at[i]], add=True)
      return carry

    lax.fori_loop(0, chunks_per_tile, body, 0)
    plsc.subcore_barrier()

    @pl.when(s < otiles)
    def _():
      pltpu.sync_copy(acc_sh.at[pl.ds(s * orows, orows)],
                      out_hbm.at[pl.ds(c * n + s * orows, orows)])

  return deg_kernel


def _make_agg_kernel(n, d_half, pad_e, acc_rows):
  chunks_per_tile = pad_e // (NS * CHUNK)
  nct = pad_e // CHUNK
  ngrp = chunks_per_tile // GRP
  assert chunks_per_tile % (4 * GRP) == 0
  zrows = acc_rows // NS
  orows = 1000
  otiles = n // orows

  @functools.partial(
      pl.kernel,
      out_type=jax.ShapeDtypeStruct((NC * n, d_half), jnp.float32),
      mesh=_sc_mesh(),
      scratch_types=[
          pltpu.VMEM((4 * GRP, CHUNK), jnp.int32),
          pltpu.VMEM((4 * GRP, CHUNK), jnp.int32),
          pltpu.VMEM((2, CHUNK, d_half), jnp.float32),
          pltpu.VMEM_SHARED((acc_rows, d_half), jnp.float32),
      ] + [pltpu.SemaphoreType.DMA] * 12,
  )
  def agg_kernel(table_hbm, src2_hbm, dst_hbm, zeros_hbm, out_hbm,
                 sidx_v, didx_v, rows_v, acc_sh, *sems):
    gsems = sems[0:2]
    ssems = sems[2:4]
    sisems = sems[4:8]
    disems = sems[8:12]
    c = lax.axis_index("c")
    s = lax.axis_index("s")
    sbase = c * nct + s * chunks_per_tile
    dbase = s * chunks_per_tile
    pltpu.sync_copy(zeros_hbm.at[pl.ds(s * zrows, zrows)],
                    acc_sh.at[pl.ds(s * zrows, zrows)])
    pltpu.sync_copy(src2_hbm.at[pl.ds(sbase, 2 * GRP)],
                    sidx_v.at[pl.ds(0, 2 * GRP)])
    pltpu.sync_copy(dst_hbm.at[pl.ds(dbase, 2 * GRP)],
                    didx_v.at[pl.ds(0, 2 * GRP)])
    pltpu.async_copy(table_hbm.at[sidx_v.at[0]], rows_v.at[0], gsems[0])
    plsc.subcore_barrier()

    def outer(ooo, carry):
      for qq in range(4):
        o = ooo * 4 + qq
        @pl.when(o + 2 < ngrp)
        def _():
          sl = (qq + 2) % 4
          pltpu.async_copy(src2_hbm.at[pl.ds(sbase + (o + 2) * GRP, GRP)],
                           sidx_v.at[pl.ds(sl * GRP, GRP)], sisems[sl])
          pltpu.async_copy(dst_hbm.at[pl.ds(dbase + (o + 2) * GRP, GRP)],
                           didx_v.at[pl.ds(sl * GRP, GRP)], disems[sl])

        @pl.when(jnp.logical_and(o >= 1, o + 1 < ngrp))
        def _():
          sl = (qq + 1) % 4
          pltpu.make_async_copy(src2_hbm.at[pl.ds(sbase + (o + 1) * GRP, GRP)],
                                sidx_v.at[pl.ds(sl * GRP, GRP)],
                                sisems[sl]).wait()
          pltpu.make_async_copy(dst_hbm.at[pl.ds(dbase + (o + 1) * GRP, GRP)],
                                didx_v.at[pl.ds(sl * GRP, GRP)],
                                disems[sl]).wait()

        for b4 in range(GRP):
          i = o * GRP + b4
          b = b4 & 1

          @pl.when(i > 0)
          def _():
            pltpu.make_async_copy(rows_v.at[1 - b],
                                  acc_sh.at[didx_v.at[qq * GRP + b4]],
                                  ssems[1 - b]).wait()

          @pl.when(i + 1 < chunks_per_tile)
          def _():
            pltpu.async_copy(table_hbm.at[sidx_v.at[(qq * GRP + b4 + 1) % 16]],
                             rows_v.at[1 - b], gsems[1 - b])

          pltpu.make_async_copy(table_hbm.at[sidx_v.at[qq * GRP + b4]],
                                rows_v.at[b], gsems[b]).wait()
          pltpu.async_copy(rows_v.at[b], acc_sh.at[didx_v.at[qq * GRP + b4]],
                           ssems[b], add=True)

      return carry

    lax.fori_loop(0, ngrp // 4, outer, 0)
    pltpu.make_async_copy(rows_v.at[(chunks_per_tile - 1) % 2],
                          acc_sh.at[didx_v.at[0]],
                          ssems[(chunks_per_tile - 1) % 2]).wait()
    plsc.subcore_barrier()

    @pl.when(s < otiles)
    def _():
      pltpu.sync_copy(acc_sh.at[pl.ds(s * orows, orows)],
                      out_hbm.at[pl.ds(c * n + s * orows, orows)])

  return agg_kernel


def _dinv_from_degp(degp):
  deg = degp[0, :, 0:1] + degp[1, :, 0:1] + 1.0
  return lax.rsqrt(deg)


def _split_h(u, u_ref):
  dh = u.shape[1] // NC
  for q in range(NC):
    u_ref[q] = u[:, q * dh:(q + 1) * dh]


def _cat_h(acc_ref, uin_ref):
  return jnp.concatenate([acc_ref[q] + uin_ref[q] for q in range(NC)], axis=1)


def _tc_mm_body(x_ref, w_ref, g_ref):
  g = jnp.dot(x_ref[...], w_ref[...], preferred_element_type=jnp.float32)
  _split_h(g, g_ref)


def _tc_scale_body(g_ref, degp_ref, u_ref):
  dinv = _dinv_from_degp(degp_ref[...])
  for q in range(NC):
    u_ref[q] = g_ref[q] * dinv


def _tc_mid_body(acc_ref, uin_ref, b_ref, w_ref, degp_ref, u_ref):
  dinv = _dinv_from_degp(degp_ref[...])
  h = jnp.maximum(_cat_h(acc_ref, uin_ref) * dinv + b_ref[...], 0.0)
  g = jnp.dot(h, w_ref[...], preferred_element_type=jnp.float32)
  _split_h(g * dinv, u_ref)


def _tc_last_body(acc_ref, uin_ref, b_ref, w_ref, bfc_ref, degp_ref, o_ref):
  dinv = _dinv_from_degp(degp_ref[...])
  h = jnp.maximum(_cat_h(acc_ref, uin_ref) * dinv + b_ref[...], 0.0)
  o_ref[...] = (jnp.dot(h, w_ref[...], preferred_element_type=jnp.float32)
                + bfc_ref[...])


def _row_spec(r, cols):
  return pl.BlockSpec((r, cols), lambda i: (i, 0))


def _stack_spec(lead, r, cols):
  return pl.BlockSpec((lead, r, cols), lambda i: (0, i, 0))


def _full_spec(shape):
  return pl.BlockSpec(shape, lambda i: tuple(0 for _ in shape))


def kernel(x, edge_index, W1, b1, W2, b2, W3, b3, Wfc, bfc):
  n, d_in = x.shape
  hid = W1.shape[1]
  d_half = hid // NC
  n_cls = Wfc.shape[1]
  e = edge_index.shape[1]

  slot = NC * NS * CHUNK * 4 * GRP
  pad_e = ((e + slot - 1) // slot) * slot
  acc_rows = ((n + 1 + 127) // 128) * 128
  nct = pad_e // CHUNK

  src = edge_index[0].astype(jnp.int32)
  dst = edge_index[1].astype(jnp.int32)
  pad = pad_e - e
  src = jnp.concatenate([src, jnp.zeros((pad,), jnp.int32)])
  dst = jnp.concatenate([dst, jnp.full((pad,), n, jnp.int32)])
  src2 = (src[None, :] + (jnp.arange(NC, dtype=jnp.int32) * n)[:, None])
  src2 = src2.reshape(NC * nct, CHUNK)
  dst2 = dst.reshape(nct, CHUNK)

  ones_rows = jnp.ones((CHUNK, d_half), jnp.float32)
  zeros_acc = jnp.zeros((acc_rows, d_half), jnp.float32)

  deg_kernel = _make_deg_kernel(n, d_half, pad_e, acc_rows)
  agg_kernel = _make_agg_kernel(n, d_half, pad_e, acc_rows)

  degp = deg_kernel(dst2, ones_rows, zeros_acc)
  degp = degp.reshape(NC, n, d_half)

  r = 1000
  grid = (n // r,)

  g1 = pl.pallas_call(
      _tc_mm_body,
      grid=grid,
      in_specs=[_row_spec(r, d_in), _full_spec((d_in, hid))],
      out_specs=_stack_spec(NC, r, d_half),
      out_shape=jax.ShapeDtypeStruct((NC, n, d_half), jnp.float32),
  )(x, W1)
  u1 = pl.pallas_call(
      _tc_scale_body,
      grid=grid,
      in_specs=[_stack_spec(NC, r, d_half), _stack_spec(NC, r, d_half)],
      out_specs=_stack_spec(NC, r, d_half),
      out_shape=jax.ShapeDtypeStruct((NC, n, d_half), jnp.float32),
  )(g1, degp)

  def mid(u_prev, b_prev, w_next):
    acc = agg_kernel(u_prev.reshape(NC * n, d_half), src2, dst2, zeros_acc)
    return pl.pallas_call(
        _tc_mid_body,
        grid=grid,
        in_specs=[_stack_spec(NC, r, d_half), _stack_spec(NC, r, d_half),
                  _full_spec((1, hid)), _full_spec((hid, hid)),
                  _stack_spec(NC, r, d_half)],
        out_specs=_stack_spec(NC, r, d_half),
        out_shape=jax.ShapeDtypeStruct((NC, n, d_half), jnp.float32),
    )(acc.reshape(NC, n, d_half), u_prev, b_prev.reshape(1, hid), w_next,
      degp)

  u2 = mid(u1, b1, W2)
  u3 = mid(u2, b2, W3)

  acc3 = agg_kernel(u3.reshape(NC * n, d_half), src2, dst2, zeros_acc)
  out = pl.pallas_call(
      _tc_last_body,
      grid=grid,
      in_specs=[_stack_spec(NC, r, d_half), _stack_spec(NC, r, d_half),
                _full_spec((1, hid)), _full_spec((hid, n_cls)),
                _full_spec((1, n_cls)), _stack_spec(NC, r, d_half)],
      out_specs=_row_spec(r, n_cls),
      out_shape=jax.ShapeDtypeStruct((n, n_cls), jnp.float32),
  )(acc3.reshape(NC, n, d_half), u3, b3.reshape(1, hid), Wfc,
    bfc.reshape(1, n_cls), degp)
  return out
